# Initial kernel scaffold; baseline (speedup 1.0000x reference)
import jax
import jax.numpy as jnp
from jax.experimental import pallas as pl


def kernel(x, edge_index, batch, W1, b1, W2, att_src, att_dst, b2, Wg, bg, Wf1, bf1, Wf2, bf2, Wo, bo):
    raise NotImplementedError("write your pallas kernel here")


# speedup vs baseline: 28.4169x; 28.4169x over previous
import functools

import jax
import jax.numpy as jnp
from jax import lax
from jax.experimental import pallas as pl
from jax.experimental.pallas import tpu as pltpu
from jax.experimental.pallas import tpu_sc as plsc

N = 10000
E = 640000
F_IN = 78
H1 = 156
HEADS = 2
OUT_DIM = 128
NGRAPH = 128

NC = 2
NS = 16
NW = NC * NS

CH = 128
EP = ((E + NW * CH - 1) // (NW * CH)) * (NW * CH)
EPT = EP // NW
NCHUNK = EPT // CH
NP = ((N + 16 + 255) // 256) * 256
NPT = NP // NS
F1P = 160
FH = F1P // 2
FP2 = 320
NPOOL = NP // NW
POOL_CH = 64
BLK = 1024
NBLK = NP // BLK

_mesh = plsc.VectorSubcoreMesh(core_axis_name="c", subcore_axis_name="s")
_params = pltpu.CompilerParams(use_tc_tiling_on_sc=False,
                               needs_layout_passes=False)


def _leaky(v, s):
    return jnp.where(v > 0, v, s * v)


def _fill(ref, n16, value, dtype=jnp.float32):
    def body(i, _):
        ref[pl.ds(i * 16, 16)] = jnp.full((16,), value, dtype)
        return 0
    lax.fori_loop(0, n16, body, 0)


@functools.partial(
    pl.kernel,
    out_type=jax.ShapeDtypeStruct((NC * NP,), jnp.float32),
    mesh=_mesh,
    compiler_params=_params,
    scratch_types=[
        pltpu.VMEM((CH,), jnp.int32),
        pltpu.VMEM((CH,), jnp.float32),
        pltpu.VMEM((NPT,), jnp.float32),
        pltpu.VMEM_SHARED((NP,), jnp.float32),
    ],
)
def _sc_deg(dst_hbm, out_hbm, idx_v, ones_v, zv, acc_sh):
    cid = lax.axis_index("c")
    sid = lax.axis_index("s")
    we = sid * NC + cid
    _fill(ones_v, CH // 16, 1.0)
    _fill(zv, NPT // 16, 0.0)
    pltpu.sync_copy(zv, acc_sh.at[pl.ds(sid * NPT, NPT)])
    plsc.subcore_barrier()

    def body(k, _):
        base = we * EPT + k * CH
        pltpu.sync_copy(dst_hbm.at[pl.ds(base, CH)], idx_v)
        pltpu.sync_copy(ones_v, acc_sh.at[idx_v], add=True)
        return 0

    lax.fori_loop(0, NCHUNK, body, 0)
    plsc.subcore_barrier()
    pltpu.sync_copy(acc_sh.at[pl.ds(sid * NPT, NPT)],
                    out_hbm.at[pl.ds(cid * NP + sid * NPT, NPT)])


@functools.partial(
    pl.kernel,
    out_type=jax.ShapeDtypeStruct((NC, NP, FH), jnp.float32),
    mesh=_mesh,
    compiler_params=_params,
    scratch_types=[
        pltpu.VMEM((CH,), jnp.int32),
        pltpu.VMEM((CH,), jnp.int32),
        pltpu.VMEM((CH, FH), jnp.float32),
        pltpu.VMEM_SHARED((NP, FH), jnp.float32),
        pltpu.SemaphoreType.DMA,
    ],
)
def _sc_gcn(src_hbm, dst_hbm, y_hbm, out_hbm,
            src_v, dst_v, rows_v, acc_sh, sem):
    cid = lax.axis_index("c")
    sid = lax.axis_index("s")
    we = sid * NC + cid

    def zrow(i, _):
        for kk in range(FH // 16):
            rows_v[i, pl.ds(kk * 16, 16)] = jnp.zeros((16,), jnp.float32)
        return 0

    lax.fori_loop(0, CH, zrow, 0)
    for i in range(NPT // CH):
        pltpu.sync_copy(rows_v, acc_sh.at[pl.ds(sid * NPT + i * CH, CH), :])
    plsc.subcore_barrier()

    def body(k, _):
        base = we * EPT + k * CH
        pltpu.sync_copy(src_hbm.at[pl.ds(base, CH)], src_v)
        pltpu.sync_copy(dst_hbm.at[pl.ds(base, CH)], dst_v)
        pltpu.async_copy(y_hbm.at[src_v], rows_v, sem).wait()
        pltpu.sync_copy(rows_v, acc_sh.at[dst_v], add=True)
        return 0

    lax.fori_loop(0, NCHUNK, body, 0)
    plsc.subcore_barrier()
    pltpu.sync_copy(acc_sh.at[pl.ds(sid * NPT, NPT), :],
                    out_hbm.at[cid, pl.ds(sid * NPT, NPT), :])


@functools.partial(
    pl.kernel,
    out_type=[
        jax.ShapeDtypeStruct((NC, NP, FH), jnp.float32),
        jax.ShapeDtypeStruct((NC * NP,), jnp.float32),
    ],
    mesh=_mesh,
    compiler_params=_params,
    scratch_types=[
        pltpu.VMEM((NP,), jnp.float32),
        pltpu.VMEM((NP,), jnp.float32),
        pltpu.VMEM((NP,), jnp.float32),
        pltpu.VMEM((CH,), jnp.int32),
        pltpu.VMEM((CH,), jnp.int32),
        pltpu.VMEM((CH + 16,), jnp.float32),
        pltpu.VMEM((CH, FH), jnp.float32),
        pltpu.VMEM((NPT,), jnp.float32),
        pltpu.VMEM_SHARED((NP, FH), jnp.float32),
        pltpu.VMEM_SHARED((NP,), jnp.float32),
        pltpu.SemaphoreType.DMA,
    ],
)
def _sc_gat(src_hbm, dst_hbm, hw_hbm, as_hbm, ad_hbm, c_hbm,
            g_out, den_out,
            as_v, ad_v, c_v, src_v, dst_v, ee_v, rows_v, zv, acc_sh, den_sh,
            sem):
    cid = lax.axis_index("c")
    sid = lax.axis_index("s")
    we = sid * NC + cid
    pltpu.sync_copy(as_hbm, as_v)
    pltpu.sync_copy(ad_hbm, ad_v)
    pltpu.sync_copy(c_hbm, c_v)

    def zrow(i, _):
        for kk in range(FH // 16):
            rows_v[i, pl.ds(kk * 16, 16)] = jnp.zeros((16,), jnp.float32)
        return 0

    lax.fori_loop(0, CH, zrow, 0)
    _fill(zv, NPT // 16, 0.0)
    for i in range(NPT // CH):
        pltpu.sync_copy(rows_v, acc_sh.at[pl.ds(sid * NPT + i * CH, CH), :])
    pltpu.sync_copy(zv, den_sh.at[pl.ds(sid * NPT, NPT)])
    plsc.subcore_barrier()

    def body(k, _):
        base = we * EPT + k * CH
        pltpu.sync_copy(src_hbm.at[pl.ds(base, CH)], src_v)
        pltpu.sync_copy(dst_hbm.at[pl.ds(base, CH)], dst_v)
        for j in range(CH // 16):
            sl = pl.ds(j * 16, 16)
            sv = src_v[sl]
            dv = dst_v[sl]
            a = plsc.load_gather(as_v, [sv]) + plsc.load_gather(ad_v, [dv])
            e = jnp.where(a > 0, a, 0.2 * a) - plsc.load_gather(c_v, [dv])
            ee_v[sl] = jnp.exp(e)
        pltpu.sync_copy(ee_v.at[pl.ds(0, CH)], den_sh.at[dst_v], add=True)
        pltpu.async_copy(hw_hbm.at[src_v], rows_v, sem).wait()

        def scale(e_ix, _):
            w = ee_v[pl.ds(e_ix, 16)][0]
            for kk in range(FH // 16):
                cs = pl.ds(kk * 16, 16)
                rows_v[e_ix, cs] = rows_v[e_ix, cs] * w
            return 0

        lax.fori_loop(0, CH, scale, 0)
        pltpu.sync_copy(rows_v, acc_sh.at[dst_v], add=True)
        return 0

    lax.fori_loop(0, NCHUNK, body, 0)
    plsc.subcore_barrier()
    pltpu.sync_copy(acc_sh.at[pl.ds(sid * NPT, NPT), :],
                    g_out.at[cid, pl.ds(sid * NPT, NPT), :])
    pltpu.sync_copy(den_sh.at[pl.ds(sid * NPT, NPT)],
                    den_out.at[pl.ds(cid * NP + sid * NPT, NPT)])


@functools.partial(
    pl.kernel,
    out_type=jax.ShapeDtypeStruct((NW * NGRAPH * FP2,), jnp.float32),
    mesh=_mesh,
    compiler_params=_params,
    scratch_types=[
        pltpu.VMEM((NGRAPH * FP2,), jnp.float32),
        pltpu.VMEM((POOL_CH * FP2,), jnp.float32),
        pltpu.VMEM((NPOOL + 16,), jnp.int32),
    ],
)
def _sc_pool(g2_hbm, batch_hbm, out_hbm, acc_v, rows_v, bat_v):
    cid = lax.axis_index("c")
    sid = lax.axis_index("s")
    we = sid * NC + cid
    _fill(acc_v, NGRAPH * FP2 // 16, -jnp.inf)
    pltpu.sync_copy(batch_hbm.at[pl.ds(we * NPOOL, NPOOL)],
                    bat_v.at[pl.ds(0, NPOOL)])

    def chunk(ci, _):
        pltpu.sync_copy(
            g2_hbm.at[pl.ds(we * NPOOL * FP2 + ci * POOL_CH * FP2,
                            POOL_CH * FP2)],
            rows_v)

        def node(ni, _):
            b = bat_v[pl.ds(ci * POOL_CH + ni, 16)][0]
            for kk in range(FP2 // 16):
                src_sl = pl.ds(ni * FP2 + kk * 16, 16)
                dst_sl = pl.ds(b * FP2 + kk * 16, 16)
                acc_v[dst_sl] = jnp.maximum(acc_v[dst_sl], rows_v[src_sl])
            return 0

        lax.fori_loop(0, POOL_CH, node, 0)
        return 0

    lax.fori_loop(0, NPOOL // POOL_CH, chunk, 0)
    pltpu.sync_copy(acc_v, out_hbm.at[pl.ds(we * NGRAPH * FP2, NGRAPH * FP2)])


def _tc_call(body, out_shape, *args):
    return pl.pallas_call(body, out_shape=out_shape)(*args)


def _pad2(a, rows, cols, value=0.0):
    r, c = a.shape
    if c < cols:
        a = jnp.concatenate([a, jnp.full((r, cols - c), value, a.dtype)], axis=1)
    if r < rows:
        a = jnp.concatenate([a, jnp.full((rows - r, cols), value, a.dtype)], axis=0)
    return a


def _tc_xw_body(x_ref, w_ref, o_ref):
    o_ref[...] = jnp.dot(x_ref[...], w_ref[...], preferred_element_type=jnp.float32, precision=lax.Precision.HIGHEST)


def _tc_y_body(degp_ref, xw_ref, yp_ref, dinv_ref):
    deg = degp_ref[0:N] + degp_ref[NP:NP + N] + 1.0
    dinv = lax.rsqrt(deg)
    y = xw_ref[...] * dinv[:, None]
    yp_ref[...] = _pad2(y, NP, F1P)
    dinv_ref[...] = jnp.concatenate(
        [dinv, jnp.zeros((NP - N,), jnp.float32)])


def _tc_hw_body(hplo_ref, hphi_ref, yp_ref, dinv_ref, b1_ref, w2_ref,
                asrc_ref, adst_ref,
                hw0_ref, hw1_ref, as0_ref, as1_ref, ad0_ref, ad1_ref):
    y = yp_ref[...][:, 0:H1]
    agg = jnp.concatenate(
        [hplo_ref[0] + hplo_ref[1],
         (hphi_ref[0] + hphi_ref[1])[:, 0:H1 - FH]], axis=1)
    h = dinv_ref[...][:, None] * (agg + y) + b1_ref[...][None, :]
    h = _leaky(h, 0.01)
    hw = jnp.dot(h, w2_ref[...], preferred_element_type=jnp.float32, precision=lax.Precision.HIGHEST)
    hw3 = hw.reshape(BLK, HEADS, H1)
    a_s = jnp.sum(hw3 * asrc_ref[...][None], axis=-1)
    a_d = jnp.sum(hw3 * adst_ref[...][None], axis=-1)
    hw0_ref[...] = _pad2(hw3[:, 0, :], BLK, F1P)
    hw1_ref[...] = _pad2(hw3[:, 1, :], BLK, F1P)
    as0_ref[...] = a_s[:, 0]
    as1_ref[...] = a_s[:, 1]
    ad0_ref[...] = a_d[:, 0]
    ad1_ref[...] = a_d[:, 1]


def _tc_logits_body(as0_ref, as1_ref, ad0_ref, ad1_ref,
                    c0_ref, c1_ref, ee_ref):
    for h, (a_s_ref, a_d_ref, c_ref) in enumerate(
            ((as0_ref, ad0_ref, c0_ref), (as1_ref, ad1_ref, c1_ref))):
        a_s = a_s_ref[...]
        a_d = a_d_ref[...]
        maxS = jnp.max(a_s[0:N])
        cm = jnp.maximum(a_d + maxS, 0.0)
        c_ref[...] = cm
        zl = a_s + a_d
        ee = jnp.exp(_leaky(zl, 0.2) - cm)
        ee_ref[h, :] = ee


def _tc_g2_body(g0l_ref, g0h_ref, d0_ref, d0b_ref, g1l_ref, g1h_ref, d1_ref,
                d1b_ref, ee_ref, hw0_ref, hw1_ref, b2_ref, g2_ref):
    parts = []
    for h, (gl_ref, gh_ref, d_ref, db_ref, hwp) in enumerate(
            ((g0l_ref, g0h_ref, d0_ref, d0b_ref, hw0_ref),
             (g1l_ref, g1h_ref, d1_ref, d1b_ref, hw1_ref))):
        eeh = ee_ref[h]
        agg = jnp.concatenate(
            [gl_ref[0] + gl_ref[1],
             (gh_ref[0] + gh_ref[1])[:, 0:H1 - FH]], axis=1)
        num = agg + eeh[:, None] * hwp[...][:, 0:H1]
        dsum = 0.5 * (d_ref[0] + d_ref[1] + db_ref[0] + db_ref[1])
        den = dsum + eeh + 1e-16
        parts.append(num / den[:, None])
    g = jnp.concatenate(parts, axis=1) + b2_ref[...][None, :]
    g2 = _pad2(_leaky(g, 0.01), BLK, FP2, -jnp.inf)
    rows = (pl.program_id(0) * BLK
            + lax.broadcasted_iota(jnp.int32, (BLK, FP2), 0))
    g2_ref[...] = jnp.where(rows < N, g2, -jnp.inf)


def _tc_head_body(pp_ref, wg_ref, bg_ref, wf1_ref, bf1_ref, wf2_ref, bf2_ref,
                  wo_ref, bo_ref, o_ref):
    p = jnp.max(pp_ref[...], axis=0)[:, 0:HEADS * H1]
    p = jnp.where(jnp.isfinite(p), p, 0.0)
    z = _leaky(jnp.dot(p, wg_ref[...], preferred_element_type=jnp.float32, precision=lax.Precision.HIGHEST)
               + bg_ref[...][None, :], 0.01)
    z = jnp.maximum(jnp.dot(z, wf1_ref[...], preferred_element_type=jnp.float32, precision=lax.Precision.HIGHEST)
                    + bf1_ref[...][None, :], 0.0)
    z = jnp.maximum(jnp.dot(z, wf2_ref[...], preferred_element_type=jnp.float32, precision=lax.Precision.HIGHEST)
                    + bf2_ref[...][None, :], 0.0)
    o_ref[...] = (jnp.dot(z, wo_ref[...], preferred_element_type=jnp.float32, precision=lax.Precision.HIGHEST)
                  + bo_ref[...][None, :])


def kernel(x, edge_index, batch, W1, b1, W2, att_src, att_dst, b2,
           Wg, bg, Wf1, bf1, Wf2, bf2, Wo, bo):
    src = edge_index[0]
    dst = edge_index[1]
    pad_ix = (N + (jnp.arange(EP - E, dtype=jnp.int32) % 16)).astype(jnp.int32)
    src_p = jnp.concatenate([src, pad_ix])
    dst_p = jnp.concatenate([dst, pad_ix])

    degp = _sc_deg(dst_p)
    xw = _tc_call(_tc_xw_body, jax.ShapeDtypeStruct((N, H1), jnp.float32), x, W1)

    yp, dinv = _tc_call(
        _tc_y_body,
        (jax.ShapeDtypeStruct((NP, F1P), jnp.float32),
         jax.ShapeDtypeStruct((NP,), jnp.float32)),
        degp, xw)

    hp_lo = _sc_gcn(src_p, dst_p, yp[:, :FH])
    hp_hi = _sc_gcn(src_p, dst_p, yp[:, FH:])

    full = lambda shp: pl.BlockSpec(shp, lambda i: tuple(0 for _ in shp))
    hw0, hw1, as0, as1, ad0, ad1 = pl.pallas_call(
        _tc_hw_body,
        grid=(NBLK,),
        in_specs=[
            pl.BlockSpec((NC, BLK, FH), lambda i: (0, i, 0)),
            pl.BlockSpec((NC, BLK, FH), lambda i: (0, i, 0)),
            pl.BlockSpec((BLK, F1P), lambda i: (i, 0)),
            pl.BlockSpec((BLK,), lambda i: (i,)),
            full((H1,)),
            full((H1, HEADS * H1)),
            full((HEADS, H1)),
            full((HEADS, H1)),
        ],
        out_specs=[
            pl.BlockSpec((BLK, F1P), lambda i: (i, 0)),
            pl.BlockSpec((BLK, F1P), lambda i: (i, 0)),
            pl.BlockSpec((BLK,), lambda i: (i,)),
            pl.BlockSpec((BLK,), lambda i: (i,)),
            pl.BlockSpec((BLK,), lambda i: (i,)),
            pl.BlockSpec((BLK,), lambda i: (i,)),
        ],
        out_shape=(jax.ShapeDtypeStruct((NP, F1P), jnp.float32),
                   jax.ShapeDtypeStruct((NP, F1P), jnp.float32),
                   jax.ShapeDtypeStruct((NP,), jnp.float32),
                   jax.ShapeDtypeStruct((NP,), jnp.float32),
                   jax.ShapeDtypeStruct((NP,), jnp.float32),
                   jax.ShapeDtypeStruct((NP,), jnp.float32)),
    )(hp_lo, hp_hi, yp, dinv, b1, W2, att_src, att_dst)

    c0, c1, ee2 = _tc_call(
        _tc_logits_body,
        (jax.ShapeDtypeStruct((NP,), jnp.float32),
         jax.ShapeDtypeStruct((NP,), jnp.float32),
         jax.ShapeDtypeStruct((HEADS, NP), jnp.float32)),
        as0, as1, ad0, ad1)

    g0l, d0 = _sc_gat(src_p, dst_p, hw0[:, :FH], as0, ad0, c0)
    g0h, d0b = _sc_gat(src_p, dst_p, hw0[:, FH:], as0, ad0, c0)
    g1l, d1 = _sc_gat(src_p, dst_p, hw1[:, :FH], as1, ad1, c1)
    g1h, d1b = _sc_gat(src_p, dst_p, hw1[:, FH:], as1, ad1, c1)

    d0r = d0.reshape(NC, NP)
    d0br = d0b.reshape(NC, NP)
    d1r = d1.reshape(NC, NP)
    d1br = d1b.reshape(NC, NP)
    g2_2d = pl.pallas_call(
        _tc_g2_body,
        grid=(NBLK,),
        in_specs=[
            pl.BlockSpec((NC, BLK, FH), lambda i: (0, i, 0)),
            pl.BlockSpec((NC, BLK, FH), lambda i: (0, i, 0)),
            pl.BlockSpec((NC, BLK), lambda i: (0, i)),
            pl.BlockSpec((NC, BLK), lambda i: (0, i)),
            pl.BlockSpec((NC, BLK, FH), lambda i: (0, i, 0)),
            pl.BlockSpec((NC, BLK, FH), lambda i: (0, i, 0)),
            pl.BlockSpec((NC, BLK), lambda i: (0, i)),
            pl.BlockSpec((NC, BLK), lambda i: (0, i)),
            pl.BlockSpec((HEADS, BLK), lambda i: (0, i)),
            pl.BlockSpec((BLK, F1P), lambda i: (i, 0)),
            pl.BlockSpec((BLK, F1P), lambda i: (i, 0)),
            full((HEADS * H1,)),
        ],
        out_specs=pl.BlockSpec((BLK, FP2), lambda i: (i, 0)),
        out_shape=jax.ShapeDtypeStruct((NP, FP2), jnp.float32),
    )(g0l, g0h, d0r, d0br, g1l, g1h, d1r, d1br, ee2, hw0, hw1, b2)
    g2 = g2_2d.reshape(NP * FP2)

    batch_p = jnp.concatenate(
        [batch.astype(jnp.int32),
         jnp.full((NP - N,), NGRAPH - 1, jnp.int32)])
    pool = _sc_pool(g2, batch_p)

    out = _tc_call(
        _tc_head_body, jax.ShapeDtypeStruct((NGRAPH, 1), jnp.float32),
        pool.reshape(NW, NGRAPH, FP2), Wg, bg, Wf1, bf1, Wf2, bf2, Wo, bo)
    return out


# speedup vs baseline: 59.5788x; 2.0966x over previous
import functools

import jax
import jax.numpy as jnp
from jax import lax
from jax.experimental import pallas as pl
from jax.experimental.pallas import tpu as pltpu
from jax.experimental.pallas import tpu_sc as plsc

N = 10000
E = 640000
F_IN = 78
H1 = 156
HEADS = 2
OUT_DIM = 128
NGRAPH = 128

NC = 2
NS = 16
NW = NC * NS

CH = 128
NCHUNK = 160
EPT = NCHUNK * CH
EP = NW * EPT
NP = ((N + 16 + 255) // 256) * 256
NPT = NP // NS
F1P = 160
FH = F1P // 2
FP2 = 320
NPOOL = NP // NW
POOL_CH = 64
BLK = 1024
NBLK = NP // BLK

_mesh = plsc.VectorSubcoreMesh(core_axis_name="c", subcore_axis_name="s")
_params = pltpu.CompilerParams(use_tc_tiling_on_sc=False,
                               needs_layout_passes=False)


def _leaky(v, s):
    return jnp.where(v > 0, v, s * v)


def _fill(ref, n16, value, dtype=jnp.float32):
    def body(i, _):
        ref[pl.ds(i * 16, 16)] = jnp.full((16,), value, dtype)
        return 0
    lax.fori_loop(0, n16, body, 0)


@functools.partial(
    pl.kernel,
    out_type=jax.ShapeDtypeStruct((NC * NP,), jnp.float32),
    mesh=_mesh,
    compiler_params=_params,
    scratch_types=[
        pltpu.VMEM((CH,), jnp.int32),
        pltpu.VMEM((CH,), jnp.float32),
        pltpu.VMEM((NPT,), jnp.float32),
        pltpu.VMEM_SHARED((NP,), jnp.float32),
    ],
)
def _sc_deg(dst_hbm, out_hbm, idx_v, ones_v, zv, acc_sh):
    cid = lax.axis_index("c")
    sid = lax.axis_index("s")
    we = sid * NC + cid
    _fill(ones_v, CH // 16, 1.0)
    _fill(zv, NPT // 16, 0.0)
    pltpu.sync_copy(zv, acc_sh.at[pl.ds(sid * NPT, NPT)])
    plsc.subcore_barrier()

    def body(k, _):
        base = we * EPT + k * CH
        pltpu.sync_copy(dst_hbm.at[pl.ds(base, CH)], idx_v)
        pltpu.sync_copy(ones_v, acc_sh.at[idx_v], add=True)
        return 0

    lax.fori_loop(0, NCHUNK, body, 0)
    plsc.subcore_barrier()
    pltpu.sync_copy(acc_sh.at[pl.ds(sid * NPT, NPT)],
                    out_hbm.at[pl.ds(cid * NP + sid * NPT, NPT)])


IB = 80
NB = NCHUNK // IB


def _make_edge_kernel(attn):
    out_type = [jax.ShapeDtypeStruct((NC, NP, FH), jnp.float32)]
    scratch = [
        pltpu.VMEM((IB, CH), jnp.int32),
        pltpu.VMEM((IB, CH), jnp.int32),
        pltpu.VMEM((CH, FH), jnp.float32),
        pltpu.VMEM((CH, FH), jnp.float32),
        pltpu.VMEM((CH, FH), jnp.float32),
        pltpu.VMEM((CH, FH), jnp.float32),
        pltpu.VMEM_SHARED((NP, FH), jnp.float32),
        pltpu.SemaphoreType.DMA,
        pltpu.SemaphoreType.DMA,
        pltpu.SemaphoreType.DMA,
        pltpu.SemaphoreType.DMA,
    ]
    if attn:
        out_type.append(jax.ShapeDtypeStruct((NC * NP,), jnp.float32))
        scratch += [
            pltpu.VMEM((CH,), jnp.float32),
            pltpu.VMEM((CH,), jnp.float32),
            pltpu.VMEM((CH,), jnp.float32),
            pltpu.VMEM((CH,), jnp.float32),
            pltpu.VMEM((CH,), jnp.float32),
            pltpu.VMEM((CH,), jnp.float32),
            pltpu.VMEM((CH,), jnp.float32),
            pltpu.VMEM((CH,), jnp.float32),
            pltpu.VMEM((NPT,), jnp.float32),
            pltpu.VMEM_SHARED((NP,), jnp.float32),
            pltpu.SemaphoreType.DMA,
            pltpu.SemaphoreType.DMA,
            pltpu.SemaphoreType.DMA,
            pltpu.SemaphoreType.DMA,
        ]

    def body(*refs):
        if attn:
            (src_hbm, dst_hbm, t_hbm, as_hbm, ad_hbm, c_hbm, g_out, den_out,
             src2_v, dst2_v, g0, g1, s0, s1, acc_sh,
             semg0, semg1, sems0, sems1,
             ab0, ab1, db0, db1, cb0, cb1, e0, e1, zv, den_sh,
             sema0, sema1, semd0, semd1) = refs
            abuf = (ab0, ab1)
            dbuf = (db0, db1)
            cbuf = (cb0, cb1)
            ering = (e0, e1)
            sema = (sema0, sema1)
            semd = (semd0, semd1)
        else:
            (src_hbm, dst_hbm, t_hbm, g_out,
             src2_v, dst2_v, g0, g1, s0, s1, acc_sh,
             semg0, semg1, sems0, sems1) = refs
        gbuf = (g0, g1)
        sbuf = (s0, s1)
        semg = (semg0, semg1)
        sems = (sems0, sems1)
        cid = lax.axis_index("c")
        sid = lax.axis_index("s")
        we = sid * NC + cid
        if attn:
            _fill(zv, NPT // 16, 0.0)
            pltpu.sync_copy(zv, den_sh.at[pl.ds(sid * NPT, NPT)])

        def zrow(i, _):
            for kk in range(FH // 16):
                s0[i, pl.ds(kk * 16, 16)] = jnp.zeros((16,), jnp.float32)
            return 0

        lax.fori_loop(0, CH, zrow, 0)
        for i in range(NPT // CH):
            pltpu.sync_copy(s0, acc_sh.at[pl.ds(sid * NPT + i * CH, CH), :])
        plsc.subcore_barrier()

        def issue(b, kl):
            pltpu.async_copy(t_hbm.at[src2_v.at[kl]], gbuf[b], semg[b])
            if attn:
                pltpu.async_copy(as_hbm.at[src2_v.at[kl]], abuf[b], sema[b])
                pltpu.async_copy(ad_hbm.at[dst2_v.at[kl]], dbuf[b], sema[b])
                pltpu.async_copy(c_hbm.at[dst2_v.at[kl]], cbuf[b], sema[b])

        def block(m, _):
            pltpu.sync_copy(
                src_hbm.at[pl.ds(we * NCHUNK + m * IB, IB), :], src2_v)
            pltpu.sync_copy(
                dst_hbm.at[pl.ds(we * NCHUNK + m * IB, IB), :], dst2_v)
            for b in range(2):
                issue(b, b)

            def chunk(k2, _):
                for b in range(2):
                    kl = 2 * k2 + b
                    g = m * IB + kl
                    pltpu.make_async_copy(
                        t_hbm.at[src2_v.at[0]], gbuf[b], semg[b]).wait()
                    if attn:
                        pltpu.make_async_copy(
                            as_hbm.at[src2_v.at[0]], abuf[b], sema[b]).wait()
                        pltpu.make_async_copy(
                            as_hbm.at[src2_v.at[0]], dbuf[b], sema[b]).wait()
                        pltpu.make_async_copy(
                            as_hbm.at[src2_v.at[0]], cbuf[b], sema[b]).wait()

                        @pl.when(g >= 2)
                        def _():
                            pltpu.make_async_copy(
                                ering[b], den_sh.at[dst2_v.at[0]],
                                semd[b]).wait()
                        for j in range(CH // 16):
                            sl = pl.ds(j * 16, 16)
                            a = abuf[b][sl] + dbuf[b][sl]
                            e = jnp.where(a > 0, a, 0.2 * a) - cbuf[b][sl]
                            ering[b][sl] = jnp.exp(e)
                        pltpu.async_copy(ering[b], den_sh.at[dst2_v.at[kl]],
                                         semd[b], add=True)

                    @pl.when(g >= 2)
                    def _():
                        pltpu.make_async_copy(
                            sbuf[b], acc_sh.at[dst2_v.at[0]], sems[b]).wait()

                    if attn:
                        def scale(e_ix, _):
                            w = ering[b][pl.ds(e_ix, 16)][0]
                            for kk in range(FH // 16):
                                cs = pl.ds(kk * 16, 16)
                                sbuf[b][e_ix, cs] = gbuf[b][e_ix, cs] * w
                            return 0
                    else:
                        def scale(e_ix, _):
                            for kk in range(FH // 16):
                                cs = pl.ds(kk * 16, 16)
                                sbuf[b][e_ix, cs] = gbuf[b][e_ix, cs]
                            return 0

                    lax.fori_loop(0, CH, scale, 0)

                    @pl.when(kl + 2 < IB)
                    def _():
                        issue(b, kl + 2)

                    pltpu.async_copy(sbuf[b], acc_sh.at[dst2_v.at[kl]],
                                     sems[b], add=True)
                return 0

            lax.fori_loop(0, IB // 2, chunk, 0)
            return 0

        lax.fori_loop(0, NB, block, 0)
        for b in range(2):
            pltpu.make_async_copy(
                sbuf[b], acc_sh.at[dst2_v.at[0]], sems[b]).wait()
            if attn:
                pltpu.make_async_copy(
                    ering[b], den_sh.at[dst2_v.at[0]], semd[b]).wait()
        plsc.subcore_barrier()
        pltpu.sync_copy(acc_sh.at[pl.ds(sid * NPT, NPT), :],
                        g_out.at[cid, pl.ds(sid * NPT, NPT), :])
        if attn:
            pltpu.sync_copy(den_sh.at[pl.ds(sid * NPT, NPT)],
                            den_out.at[pl.ds(cid * NP + sid * NPT, NPT)])

    return functools.partial(
        pl.kernel, mesh=_mesh, compiler_params=_params,
        out_type=out_type if attn else out_type[0],
        scratch_types=scratch)(body)


_sc_gcn = _make_edge_kernel(False)
_sc_gat = _make_edge_kernel(True)


@functools.partial(
    pl.kernel,
    out_type=jax.ShapeDtypeStruct((NW * NGRAPH * FP2,), jnp.float32),
    mesh=_mesh,
    compiler_params=_params,
    scratch_types=[
        pltpu.VMEM((NGRAPH * FP2,), jnp.float32),
        pltpu.VMEM((POOL_CH * FP2,), jnp.float32),
        pltpu.VMEM((NPOOL + 16,), jnp.int32),
    ],
)
def _sc_pool(g2_hbm, batch_hbm, out_hbm, acc_v, rows_v, bat_v):
    cid = lax.axis_index("c")
    sid = lax.axis_index("s")
    we = sid * NC + cid
    _fill(acc_v, NGRAPH * FP2 // 16, -jnp.inf)
    pltpu.sync_copy(batch_hbm.at[pl.ds(we * NPOOL, NPOOL)],
                    bat_v.at[pl.ds(0, NPOOL)])

    def chunk(ci, _):
        pltpu.sync_copy(
            g2_hbm.at[pl.ds(we * NPOOL * FP2 + ci * POOL_CH * FP2,
                            POOL_CH * FP2)],
            rows_v)

        def node(ni, _):
            b = bat_v[pl.ds(ci * POOL_CH + ni, 16)][0]
            for kk in range(FP2 // 16):
                src_sl = pl.ds(ni * FP2 + kk * 16, 16)
                dst_sl = pl.ds(b * FP2 + kk * 16, 16)
                acc_v[dst_sl] = jnp.maximum(acc_v[dst_sl], rows_v[src_sl])
            return 0

        lax.fori_loop(0, POOL_CH, node, 0)
        return 0

    lax.fori_loop(0, NPOOL // POOL_CH, chunk, 0)
    pltpu.sync_copy(acc_v, out_hbm.at[pl.ds(we * NGRAPH * FP2, NGRAPH * FP2)])


def _tc_call(body, out_shape, *args):
    return pl.pallas_call(body, out_shape=out_shape)(*args)


def _pad2(a, rows, cols, value=0.0):
    r, c = a.shape
    if c < cols:
        a = jnp.concatenate([a, jnp.full((r, cols - c), value, a.dtype)], axis=1)
    if r < rows:
        a = jnp.concatenate([a, jnp.full((rows - r, cols), value, a.dtype)], axis=0)
    return a


def _tc_xw_body(x_ref, w_ref, o_ref):
    o_ref[...] = jnp.dot(x_ref[...], w_ref[...], preferred_element_type=jnp.float32, precision=lax.Precision.HIGHEST)


def _tc_y_body(degp_ref, xw_ref, yp_ref, dinv_ref):
    deg = degp_ref[0:N] + degp_ref[NP:NP + N] + 1.0
    dinv = lax.rsqrt(deg)
    y = xw_ref[...] * dinv[:, None]
    yp_ref[...] = _pad2(y, NP, F1P)
    dinv_ref[...] = jnp.concatenate(
        [dinv, jnp.zeros((NP - N,), jnp.float32)])


def _tc_hw_body(hplo_ref, hphi_ref, yp_ref, dinv_ref, b1_ref, w2_ref,
                asrc_ref, adst_ref,
                hw0_ref, hw1_ref, as0_ref, as1_ref, ad0_ref, ad1_ref):
    y = yp_ref[...][:, 0:H1]
    agg = jnp.concatenate(
        [hplo_ref[0] + hplo_ref[1],
         (hphi_ref[0] + hphi_ref[1])[:, 0:H1 - FH]], axis=1)
    h = dinv_ref[...][:, None] * (agg + y) + b1_ref[...][None, :]
    h = _leaky(h, 0.01)
    hw = jnp.dot(h, w2_ref[...], preferred_element_type=jnp.float32, precision=lax.Precision.HIGHEST)
    hw3 = hw.reshape(BLK, HEADS, H1)
    a_s = jnp.sum(hw3 * asrc_ref[...][None], axis=-1)
    a_d = jnp.sum(hw3 * adst_ref[...][None], axis=-1)
    hw0_ref[...] = _pad2(hw3[:, 0, :], BLK, F1P)
    hw1_ref[...] = _pad2(hw3[:, 1, :], BLK, F1P)
    as0_ref[...] = a_s[:, 0]
    as1_ref[...] = a_s[:, 1]
    ad0_ref[...] = a_d[:, 0]
    ad1_ref[...] = a_d[:, 1]


def _tc_logits_body(as0_ref, as1_ref, ad0_ref, ad1_ref,
                    c0_ref, c1_ref, ee_ref):
    for h, (a_s_ref, a_d_ref, c_ref) in enumerate(
            ((as0_ref, ad0_ref, c0_ref), (as1_ref, ad1_ref, c1_ref))):
        a_s = a_s_ref[...]
        a_d = a_d_ref[...]
        maxS = jnp.max(a_s[0:N])
        cm = jnp.maximum(a_d + maxS, 0.0)
        c_ref[...] = cm
        zl = a_s + a_d
        ee = jnp.exp(_leaky(zl, 0.2) - cm)
        ee_ref[h, :] = ee


def _tc_g2_body(g0l_ref, g0h_ref, d0_ref, d0b_ref, g1l_ref, g1h_ref, d1_ref,
                d1b_ref, ee_ref, hw0_ref, hw1_ref, b2_ref, g2_ref):
    parts = []
    for h, (gl_ref, gh_ref, d_ref, db_ref, hwp) in enumerate(
            ((g0l_ref, g0h_ref, d0_ref, d0b_ref, hw0_ref),
             (g1l_ref, g1h_ref, d1_ref, d1b_ref, hw1_ref))):
        eeh = ee_ref[h]
        agg = jnp.concatenate(
            [gl_ref[0] + gl_ref[1],
             (gh_ref[0] + gh_ref[1])[:, 0:H1 - FH]], axis=1)
        num = agg + eeh[:, None] * hwp[...][:, 0:H1]
        dsum = 0.5 * (d_ref[0] + d_ref[1] + db_ref[0] + db_ref[1])
        den = dsum + eeh + 1e-16
        parts.append(num / den[:, None])
    g = jnp.concatenate(parts, axis=1) + b2_ref[...][None, :]
    g2 = _pad2(_leaky(g, 0.01), BLK, FP2, -jnp.inf)
    rows = (pl.program_id(0) * BLK
            + lax.broadcasted_iota(jnp.int32, (BLK, FP2), 0))
    g2_ref[...] = jnp.where(rows < N, g2, -jnp.inf)


def _tc_head_body(pp_ref, wg_ref, bg_ref, wf1_ref, bf1_ref, wf2_ref, bf2_ref,
                  wo_ref, bo_ref, o_ref):
    p = jnp.max(pp_ref[...], axis=0)[:, 0:HEADS * H1]
    p = jnp.where(jnp.isfinite(p), p, 0.0)
    z = _leaky(jnp.dot(p, wg_ref[...], preferred_element_type=jnp.float32, precision=lax.Precision.HIGHEST)
               + bg_ref[...][None, :], 0.01)
    z = jnp.maximum(jnp.dot(z, wf1_ref[...], preferred_element_type=jnp.float32, precision=lax.Precision.HIGHEST)
                    + bf1_ref[...][None, :], 0.0)
    z = jnp.maximum(jnp.dot(z, wf2_ref[...], preferred_element_type=jnp.float32, precision=lax.Precision.HIGHEST)
                    + bf2_ref[...][None, :], 0.0)
    o_ref[...] = (jnp.dot(z, wo_ref[...], preferred_element_type=jnp.float32, precision=lax.Precision.HIGHEST)
                  + bo_ref[...][None, :])


def kernel(x, edge_index, batch, W1, b1, W2, att_src, att_dst, b2,
           Wg, bg, Wf1, bf1, Wf2, bf2, Wo, bo):
    src = edge_index[0]
    dst = edge_index[1]
    pad_ix = (N + (jnp.arange(EP - E, dtype=jnp.int32) % 16)).astype(jnp.int32)
    src_p = jnp.concatenate([src, pad_ix]).reshape(EP // CH, CH)
    dst_p = jnp.concatenate([dst, pad_ix]).reshape(EP // CH, CH)

    degp = _sc_deg(dst_p.reshape(EP))
    xw = _tc_call(_tc_xw_body, jax.ShapeDtypeStruct((N, H1), jnp.float32), x, W1)

    yp, dinv = _tc_call(
        _tc_y_body,
        (jax.ShapeDtypeStruct((NP, F1P), jnp.float32),
         jax.ShapeDtypeStruct((NP,), jnp.float32)),
        degp, xw)

    hp_lo = _sc_gcn(src_p, dst_p, yp[:, :FH])
    hp_hi = _sc_gcn(src_p, dst_p, yp[:, FH:])

    full = lambda shp: pl.BlockSpec(shp, lambda i: tuple(0 for _ in shp))
    hw0, hw1, as0, as1, ad0, ad1 = pl.pallas_call(
        _tc_hw_body,
        grid=(NBLK,),
        in_specs=[
            pl.BlockSpec((NC, BLK, FH), lambda i: (0, i, 0)),
            pl.BlockSpec((NC, BLK, FH), lambda i: (0, i, 0)),
            pl.BlockSpec((BLK, F1P), lambda i: (i, 0)),
            pl.BlockSpec((BLK,), lambda i: (i,)),
            full((H1,)),
            full((H1, HEADS * H1)),
            full((HEADS, H1)),
            full((HEADS, H1)),
        ],
        out_specs=[
            pl.BlockSpec((BLK, F1P), lambda i: (i, 0)),
            pl.BlockSpec((BLK, F1P), lambda i: (i, 0)),
            pl.BlockSpec((BLK,), lambda i: (i,)),
            pl.BlockSpec((BLK,), lambda i: (i,)),
            pl.BlockSpec((BLK,), lambda i: (i,)),
            pl.BlockSpec((BLK,), lambda i: (i,)),
        ],
        out_shape=(jax.ShapeDtypeStruct((NP, F1P), jnp.float32),
                   jax.ShapeDtypeStruct((NP, F1P), jnp.float32),
                   jax.ShapeDtypeStruct((NP,), jnp.float32),
                   jax.ShapeDtypeStruct((NP,), jnp.float32),
                   jax.ShapeDtypeStruct((NP,), jnp.float32),
                   jax.ShapeDtypeStruct((NP,), jnp.float32)),
    )(hp_lo, hp_hi, yp, dinv, b1, W2, att_src, att_dst)

    c0, c1, ee2 = _tc_call(
        _tc_logits_body,
        (jax.ShapeDtypeStruct((NP,), jnp.float32),
         jax.ShapeDtypeStruct((NP,), jnp.float32),
         jax.ShapeDtypeStruct((HEADS, NP), jnp.float32)),
        as0, as1, ad0, ad1)

    g0l, d0 = _sc_gat(src_p, dst_p, hw0[:, :FH], as0, ad0, c0)
    g0h, d0b = _sc_gat(src_p, dst_p, hw0[:, FH:], as0, ad0, c0)
    g1l, d1 = _sc_gat(src_p, dst_p, hw1[:, :FH], as1, ad1, c1)
    g1h, d1b = _sc_gat(src_p, dst_p, hw1[:, FH:], as1, ad1, c1)

    d0r = d0.reshape(NC, NP)
    d0br = d0b.reshape(NC, NP)
    d1r = d1.reshape(NC, NP)
    d1br = d1b.reshape(NC, NP)
    g2_2d = pl.pallas_call(
        _tc_g2_body,
        grid=(NBLK,),
        in_specs=[
            pl.BlockSpec((NC, BLK, FH), lambda i: (0, i, 0)),
            pl.BlockSpec((NC, BLK, FH), lambda i: (0, i, 0)),
            pl.BlockSpec((NC, BLK), lambda i: (0, i)),
            pl.BlockSpec((NC, BLK), lambda i: (0, i)),
            pl.BlockSpec((NC, BLK, FH), lambda i: (0, i, 0)),
            pl.BlockSpec((NC, BLK, FH), lambda i: (0, i, 0)),
            pl.BlockSpec((NC, BLK), lambda i: (0, i)),
            pl.BlockSpec((NC, BLK), lambda i: (0, i)),
            pl.BlockSpec((HEADS, BLK), lambda i: (0, i)),
            pl.BlockSpec((BLK, F1P), lambda i: (i, 0)),
            pl.BlockSpec((BLK, F1P), lambda i: (i, 0)),
            full((HEADS * H1,)),
        ],
        out_specs=pl.BlockSpec((BLK, FP2), lambda i: (i, 0)),
        out_shape=jax.ShapeDtypeStruct((NP, FP2), jnp.float32),
    )(g0l, g0h, d0r, d0br, g1l, g1h, d1r, d1br, ee2, hw0, hw1, b2)
    g2 = g2_2d.reshape(NP * FP2)

    batch_p = jnp.concatenate(
        [batch.astype(jnp.int32),
         jnp.full((NP - N,), NGRAPH - 1, jnp.int32)])
    pool = _sc_pool(g2, batch_p)

    out = _tc_call(
        _tc_head_body, jax.ShapeDtypeStruct((NGRAPH, 1), jnp.float32),
        pool.reshape(NW, NGRAPH, FP2), Wg, bg, Wf1, bf1, Wf2, bf2, Wo, bo)
    return out
